# baseline (speedup 1.0000x reference)
import functools

import jax
import jax.numpy as jnp
from jax.experimental import pallas as pl
from jax.experimental.pallas import tpu as pltpu

_BN_EPS = 1e-5
_VMEM_LIMIT = 64 * 1024 * 1024


def _fold(stat_ref, inv_m, g, b):
    s = jnp.sum(stat_ref[:, :, 0:1], axis=0)
    q = jnp.sum(stat_ref[:, :, 1:2], axis=0)
    mean = s * inv_m
    var = jnp.maximum(q * inv_m - mean * mean, 0.0)
    sc = g * jax.lax.rsqrt(var + _BN_EPS)
    return sc, b - mean * sc


def _stats1_kernel(x_ref, w1_ref, h1_ref, stat_ref):
    @pl.when(pl.program_id(1) == 0)
    def _():
        stat_ref[...] = jnp.zeros_like(stat_ref)
    cin = x_ref.shape[0]
    x2 = x_ref[...].astype(jnp.bfloat16).reshape(cin, h1_ref.shape[1])
    h = jnp.dot(w1_ref[...].astype(jnp.bfloat16), x2,
                preferred_element_type=jnp.float32)
    stat_ref[...] += jnp.concatenate(
        [jnp.sum(h, axis=1, keepdims=True),
         jnp.sum(h * h, axis=1, keepdims=True)], axis=1)
    h1_ref[...] = h.astype(h1_ref.dtype)


def _stats2_kernel(h1_ref, w2_ref, s1_ref, gb_ref, stat_ref, *, inv_m):
    @pl.when(pl.program_id(1) == 0)
    def _():
        stat_ref[...] = jnp.zeros_like(stat_ref)
    sc1, sh1 = _fold(s1_ref, inv_m, gb_ref[:, 0:1], gb_ref[:, 1:2])
    a1 = jnp.maximum(h1_ref[...].astype(jnp.float32) * sc1 + sh1, 0.0)
    h2 = jnp.dot(w2_ref[...], a1, preferred_element_type=jnp.float32)
    stat_ref[...] += jnp.concatenate(
        [jnp.sum(h2, axis=1, keepdims=True),
         jnp.sum(h2 * h2, axis=1, keepdims=True)], axis=1)


def _apply_kernel(h1_ref, w2_ref, s1_ref, s2_ref, gb_ref, o_ref,
                  *, inv_m, ht, wd):
    sc1, sh1 = _fold(s1_ref, inv_m, gb_ref[:, 0:1], gb_ref[:, 1:2])
    sc2, sh2 = _fold(s2_ref, inv_m, gb_ref[:, 2:3], gb_ref[:, 3:4])
    a1 = jnp.maximum(h1_ref[...].astype(jnp.float32) * sc1 + sh1, 0.0)
    h2 = jnp.dot(w2_ref[...], a1, preferred_element_type=jnp.float32)
    o = jnp.maximum(h2 * sc2 + sh2, 0.0)
    o_ref[...] = o.reshape(o.shape[0], ht, wd)


def kernel(x, w1, w2, gamma1, beta1, gamma2, beta2):
    n, cin, h, w = x.shape
    cout = w1.shape[0]
    hw = h * w
    inv_m = 1.0 / float(n * hw)

    split = 2
    assert h % split == 0 and (h // split) * w % 128 == 0
    ht = h // split
    tile = ht * w

    gb = jnp.stack([gamma1, beta1, gamma2, beta2], axis=1)

    grid = (split, n)
    sem = ("parallel", "arbitrary")
    cp = pltpu.CompilerParams(dimension_semantics=sem,
                              vmem_limit_bytes=_VMEM_LIMIT)
    x_spec = pl.BlockSpec((None, cin, ht, w), lambda s, i: (i, 0, s, 0))
    o_spec = pl.BlockSpec((None, cout, ht, w), lambda s, i: (i, 0, s, 0))
    f_spec = pl.BlockSpec((None, cout, tile), lambda s, i: (i, 0, s))
    w_spec = lambda a, b: pl.BlockSpec((a, b), lambda s, i: (0, 0))
    stat_spec = pl.BlockSpec((None, cout, 2), lambda s, i: (s, 0, 0))
    stat_full = pl.BlockSpec((split, cout, 2), lambda s, i: (0, 0, 0))
    stat_shape = jax.ShapeDtypeStruct((split, cout, 2), jnp.float32)

    h1, stats1 = pl.pallas_call(
        _stats1_kernel,
        out_shape=(jax.ShapeDtypeStruct((n, cout, hw), jnp.bfloat16),
                   stat_shape),
        grid=grid,
        in_specs=[x_spec, w_spec(cout, cin)],
        out_specs=(f_spec, stat_spec),
        compiler_params=cp,
    )(x, w1)

    stats2 = pl.pallas_call(
        functools.partial(_stats2_kernel, inv_m=inv_m),
        out_shape=stat_shape,
        grid=grid,
        in_specs=[f_spec, w_spec(cout, cout), stat_full, w_spec(cout, 4)],
        out_specs=stat_spec,
        compiler_params=cp,
    )(h1, w2, stats1, gb)

    out = pl.pallas_call(
        functools.partial(_apply_kernel, inv_m=inv_m, ht=ht, wd=w),
        out_shape=jax.ShapeDtypeStruct((n, cout, h, w), jnp.float32),
        grid=grid,
        in_specs=[f_spec, w_spec(cout, cout), stat_full, stat_full,
                  w_spec(cout, 4)],
        out_specs=o_spec,
        compiler_params=cp,
    )(h1, w2, stats1, stats2, gb)

    return out


# speedup vs baseline: 1.2502x; 1.2502x over previous
import functools

import jax
import jax.numpy as jnp
from jax.experimental import pallas as pl
from jax.experimental.pallas import tpu as pltpu

_BN_EPS = 1e-5
_VMEM_LIMIT = 64 * 1024 * 1024


def _fused_kernel(x_ref, w1_ref, w2_ref, gb_ref, o_ref,
                  h1_scr, s1_scr, q1_scr, s2_scr, q2_scr,
                  *, n, kc, ht, wd, inv_m):
    p = pl.program_id(0)
    i = pl.program_id(1)
    c = pl.program_id(2)
    first = jnp.logical_and(i == 0, c == 0)
    tile = ht * wd

    def fold(s_scr, q_scr, g, b):
        mean = s_scr[...] * inv_m
        var = jnp.maximum(q_scr[...] * inv_m - mean * mean, 0.0)
        sc = g * jax.lax.rsqrt(var + _BN_EPS)
        return sc, b - mean * sc

    @pl.when(p == 0)
    def _():
        @pl.when(first)
        def _():
            s1_scr[...] = jnp.zeros_like(s1_scr)
            q1_scr[...] = jnp.zeros_like(q1_scr)
        x2 = x_ref[...].astype(jnp.bfloat16).reshape(x_ref.shape[0], tile)
        h = jnp.dot(w1_ref[...].astype(jnp.bfloat16), x2,
                    preferred_element_type=jnp.float32)
        s1_scr[...] += jnp.sum(h, axis=1, keepdims=True)
        q1_scr[...] += jnp.sum(h * h, axis=1, keepdims=True)
        h1_scr[i, :, pl.ds(c * tile, tile)] = h.astype(h1_scr.dtype)

    @pl.when(p == 1)
    def _():
        @pl.when(first)
        def _():
            s2_scr[...] = jnp.zeros_like(s2_scr)
            q2_scr[...] = jnp.zeros_like(q2_scr)
        sc1, sh1 = fold(s1_scr, q1_scr, gb_ref[:, 0:1], gb_ref[:, 1:2])
        h1 = h1_scr[i, :, pl.ds(c * tile, tile)]
        a1 = jnp.maximum(h1 * sc1.astype(jnp.bfloat16) + sh1.astype(jnp.bfloat16),
                         jnp.bfloat16(0.0))
        h2 = jnp.dot(w2_ref[...].astype(jnp.bfloat16), a1,
                     preferred_element_type=jnp.float32)
        s2_scr[...] += jnp.sum(h2, axis=1, keepdims=True)
        q2_scr[...] += jnp.sum(h2 * h2, axis=1, keepdims=True)

    @pl.when(p == 2)
    def _():
        sc1, sh1 = fold(s1_scr, q1_scr, gb_ref[:, 0:1], gb_ref[:, 1:2])
        sc2, sh2 = fold(s2_scr, q2_scr, gb_ref[:, 2:3], gb_ref[:, 3:4])
        h1 = h1_scr[i, :, pl.ds(c * tile, tile)]
        a1 = jnp.maximum(h1 * sc1.astype(jnp.bfloat16) + sh1.astype(jnp.bfloat16),
                         jnp.bfloat16(0.0))
        h2 = jnp.dot(w2_ref[...].astype(jnp.bfloat16), a1,
                     preferred_element_type=jnp.float32)
        o = jnp.maximum(h2 * sc2 + sh2, 0.0)
        o_ref[...] = o.reshape(o.shape[0], ht, wd)


def kernel(x, w1, w2, gamma1, beta1, gamma2, beta2):
    n, cin, h, w = x.shape
    cout = w1.shape[0]
    inv_m = 1.0 / float(n * h * w)

    kc = 2
    assert h % kc == 0 and (h // kc) * w % 128 == 0
    ht = h // kc

    gb = jnp.stack([gamma1, beta1, gamma2, beta2], axis=1)

    x_spec = pl.BlockSpec(
        (None, cin, ht, w),
        lambda p, i, c: (jnp.where(p == 0, i, n - 1), 0,
                         jnp.where(p == 0, c, kc - 1), 0))
    o_spec = pl.BlockSpec(
        (None, cout, ht, w),
        lambda p, i, c: (jnp.where(p == 2, i, 0), 0,
                         jnp.where(p == 2, c, 0), 0))
    w_spec = lambda a, b: pl.BlockSpec((a, b), lambda p, i, c: (0, 0))

    out = pl.pallas_call(
        functools.partial(_fused_kernel, n=n, kc=kc, ht=ht, wd=w, inv_m=inv_m),
        out_shape=jax.ShapeDtypeStruct((n, cout, h, w), jnp.float32),
        grid=(3, n, kc),
        in_specs=[x_spec, w_spec(cout, cin), w_spec(cout, cout), w_spec(cout, 4)],
        out_specs=o_spec,
        scratch_shapes=[
            pltpu.VMEM((n, cout, h * w), jnp.bfloat16),
            pltpu.VMEM((cout, 1), jnp.float32),
            pltpu.VMEM((cout, 1), jnp.float32),
            pltpu.VMEM((cout, 1), jnp.float32),
            pltpu.VMEM((cout, 1), jnp.float32),
        ],
        compiler_params=pltpu.CompilerParams(
            dimension_semantics=("arbitrary", "arbitrary", "arbitrary"),
            vmem_limit_bytes=_VMEM_LIMIT),
    )(x, w1, w2, gb)

    return out
